# Initial kernel scaffold; baseline (speedup 1.0000x reference)
import jax
import jax.numpy as jnp
from jax.experimental import pallas as pl


def kernel(vertices, faces):
    raise NotImplementedError("write your pallas kernel here")


# speedup vs baseline: 82.8434x; 82.8434x over previous
import functools

import jax
import jax.numpy as jnp
from jax import lax
from jax.experimental import pallas as pl
from jax.experimental.pallas import tpu as pltpu
from jax.experimental.pallas import tpu_sc as plsc

V_CHUNK = 400
F_CHUNK = 128
RW = 8


def _sc_accumulate(vertices, faces_t, zeros_row, B, V, FPAD):
    info = plsc.get_sparse_core_info()
    NC, NS, L = info.num_cores, info.num_subcores, info.num_lanes
    BH = B // NC
    faces_per_tile = FPAD // NS
    n_fchunks = faces_per_tile // F_CHUNK
    nv_chunks = V // V_CHUNK
    mesh = plsc.VectorSubcoreMesh(core_axis_name="c", subcore_axis_name="s")

    @functools.partial(
        pl.kernel,
        out_type=jax.ShapeDtypeStruct((NC, V, RW), jnp.float32),
        mesh=mesh,
        compiler_params=pltpu.CompilerParams(
            needs_layout_passes=False, use_tc_tiling_on_sc=False),
        scratch_types=[
            pltpu.VMEM_SHARED((V, RW), jnp.float32),
            pltpu.VMEM_SHARED((V, RW), jnp.float32),
            pltpu.VMEM((V_CHUNK, 3), jnp.float32),
            pltpu.VMEM((V_CHUNK, RW), jnp.float32),
            pltpu.VMEM((V_CHUNK, RW), jnp.float32),
            pltpu.VMEM((3, F_CHUNK), jnp.int32),
            pltpu.VMEM((3, F_CHUNK, RW), jnp.float32),
            pltpu.VMEM((F_CHUNK, RW), jnp.float32),
        ],
    )
    def sc_kernel(verts_hbm, faces_hbm, zeros_hbm, out_hbm,
                  uw_s, acc_s, vbuf, uwbuf, zbuf, idxbuf, gbuf, sbuf):
        cid = lax.axis_index("c")
        sid = lax.axis_index("s")
        iota = lax.iota(jnp.int32, L)

        def cfull(val):
            return jnp.full((L,), val, jnp.int32)

        pltpu.sync_copy(zeros_hbm, zbuf)

        n_mine = jnp.where(sid < (nv_chunks % NS), nv_chunks // NS + 1,
                           nv_chunks // NS).astype(jnp.int32)

        def build_body(i, _):
            c = sid + i * NS
            v0 = c * V_CHUNK
            for bb in range(BH):
                b = cid * BH + bb
                pltpu.sync_copy(verts_hbm.at[b, pl.ds(v0, V_CHUNK), :], vbuf)

                def grp(j, _):
                    r = iota + j * L
                    x = plsc.load_gather(vbuf, [r, cfull(0)])
                    y = plsc.load_gather(vbuf, [r, cfull(1)])
                    z = plsc.load_gather(vbuf, [r, cfull(2)])
                    plsc.store_scatter(uwbuf, [r, cfull(bb)], y - x)
                    plsc.store_scatter(uwbuf, [r, cfull(BH + bb)], z - x)
                    return 0
                lax.fori_loop(0, V_CHUNK // L, grp, 0)
            pltpu.sync_copy(uwbuf, uw_s.at[pl.ds(v0, V_CHUNK), :])
            pltpu.sync_copy(zbuf, acc_s.at[pl.ds(v0, V_CHUNK), :])
            return 0
        lax.fori_loop(0, n_mine, build_body, 0)

        plsc.subcore_barrier()

        def face_body(i, _):
            f0 = sid * faces_per_tile + i * F_CHUNK
            pltpu.sync_copy(faces_hbm.at[:, pl.ds(f0, F_CHUNK)], idxbuf)
            for k in range(3):
                pltpu.sync_copy(uw_s.at[idxbuf.at[k]], gbuf.at[k])
            for j in range(F_CHUNK // L):
                r = iota + j * L
                for bb in range(BH):
                    a0 = plsc.load_gather(gbuf, [cfull(0), r, cfull(bb)])
                    a1 = plsc.load_gather(gbuf, [cfull(1), r, cfull(bb)])
                    a2 = plsc.load_gather(gbuf, [cfull(2), r, cfull(bb)])
                    c0 = plsc.load_gather(gbuf, [cfull(0), r, cfull(BH + bb)])
                    c1 = plsc.load_gather(gbuf, [cfull(1), r, cfull(BH + bb)])
                    c2 = plsc.load_gather(gbuf, [cfull(2), r, cfull(BH + bb)])
                    plsc.store_scatter(sbuf, [r, cfull(bb * 3 + 0)],
                                       a1 * c2 - a2 * c1)
                    plsc.store_scatter(sbuf, [r, cfull(bb * 3 + 1)],
                                       a2 * c0 - a0 * c2)
                    plsc.store_scatter(sbuf, [r, cfull(bb * 3 + 2)],
                                       a0 * c1 - a1 * c0)
                plsc.store_scatter(sbuf, [r, cfull(6)], jnp.zeros((L,), jnp.float32))
                plsc.store_scatter(sbuf, [r, cfull(7)], jnp.zeros((L,), jnp.float32))
            for k in range(3):
                pltpu.sync_copy(sbuf, acc_s.at[idxbuf.at[k]], add=True)
            return 0
        lax.fori_loop(0, n_fchunks, face_body, 0)

        plsc.subcore_barrier()

        def wo_body(i, _):
            c = sid + i * NS
            v0 = c * V_CHUNK
            pltpu.sync_copy(acc_s.at[pl.ds(v0, V_CHUNK), :], zbuf)
            pltpu.sync_copy(zbuf, out_hbm.at[cid, pl.ds(v0, V_CHUNK), :])
            return 0
        lax.fori_loop(0, n_mine, wo_body, 0)

    return sc_kernel(vertices, faces_t, zeros_row)


def _tc_normalize(partials, B, V):
    VB = 800
    NC = partials.shape[0]
    BH = B // NC

    def body(p_ref, o_ref):
        for c in range(NC):
            s = p_ref[c]
            for bb in range(BH):
                sl = s[:, bb * 3:(bb + 1) * 3]
                nrm = jnp.sqrt(jnp.sum(sl * sl, axis=1, keepdims=True))
                o_ref[c * BH + bb] = sl / jnp.maximum(nrm, 1e-6)

    return pl.pallas_call(
        body,
        grid=(V // VB,),
        in_specs=[pl.BlockSpec((NC, VB, RW), lambda i: (0, i, 0))],
        out_specs=pl.BlockSpec((B, VB, 3), lambda i: (0, i, 0)),
        out_shape=jax.ShapeDtypeStruct((B, V, 3), jnp.float32),
    )(partials)


def kernel(vertices, faces):
    faces = jnp.squeeze(faces).astype(jnp.int32)
    B, V, _ = vertices.shape
    F = faces.shape[0]
    NS = 16
    FPAD = -(-F // (NS * F_CHUNK)) * (NS * F_CHUNK)
    faces_t = jnp.zeros((3, FPAD), jnp.int32).at[:, :F].set(faces.T)
    zeros_row = jnp.zeros((V_CHUNK, RW), jnp.float32)
    partials = _sc_accumulate(vertices, faces_t, zeros_row, B, V, FPAD)
    return _tc_normalize(partials, B, V)
